# Initial kernel scaffold; baseline (speedup 1.0000x reference)
import functools

import jax
import jax.numpy as jnp
from jax import lax
from jax.experimental import pallas as pl
from jax.experimental.pallas import tpu as pltpu
from jax.experimental.pallas import tpu_sc as plsc

N = 100000
NUM_FIELDS = 9
VOCAB = 100
HIDDEN = 128

NW = 32
BB = 80
NBLK = N // BB
BLK_PER_W = -(-NBLK // NW)


def _body(idxT_hbm, ft_hbm, out_hbm, idx_v, acc_v, stg_v, sem):
    wid = lax.axis_index("s") * 2 + lax.axis_index("c")

    def block(k, _):
        blk = k * NW + wid

        @pl.when(blk < NBLK)
        def _():
            base = blk * BB
            pltpu.sync_copy(idxT_hbm.at[:, pl.ds(base, BB)], idx_v)
            pltpu.async_copy(ft_hbm.at[idx_v.at[0]], acc_v, sem).wait()
            for j in range(1, NUM_FIELDS):
                pltpu.async_copy(ft_hbm.at[idx_v.at[j]], stg_v, sem).wait()

                def row(r, _):
                    for c in range(HIDDEN // 16):
                        sl = pl.ds(c * 16, 16)
                        acc_v[r, sl] = acc_v[r, sl] + stg_v[r, sl]
                    return 0

                lax.fori_loop(0, BB, row, 0)
            pltpu.sync_copy(acc_v, out_hbm.at[pl.ds(base, BB)])

        return 0

    lax.fori_loop(0, BLK_PER_W, block, 0)


@jax.jit
def kernel(x, tables):
    ft = tables.reshape(NUM_FIELDS * VOCAB, HIDDEN)
    offs = (jnp.arange(NUM_FIELDS, dtype=jnp.int32) * VOCAB)[:, None]
    idxT = x.astype(jnp.int32).T + offs

    mesh = plsc.VectorSubcoreMesh(core_axis_name="c", subcore_axis_name="s")
    run = pl.kernel(
        _body,
        out_type=jax.ShapeDtypeStruct((N, HIDDEN), jnp.float32),
        mesh=mesh,
        scratch_types=[
            pltpu.VMEM((NUM_FIELDS, BB), jnp.int32),
            pltpu.VMEM((BB, HIDDEN), jnp.float32),
            pltpu.VMEM((BB, HIDDEN), jnp.float32),
            pltpu.SemaphoreType.DMA,
        ],
    )
    return run(idxT, ft)


# speedup vs baseline: 2.4919x; 2.4919x over previous
import functools

import jax
import jax.numpy as jnp
from jax import lax
from jax.experimental import pallas as pl
from jax.experimental.pallas import tpu as pltpu
from jax.experimental.pallas import tpu_sc as plsc

N = 100000
NUM_FIELDS = 9
VOCAB = 100
HIDDEN = 128

NW = 32
BB = 80
NBLK = N // BB
BLK_PER_W = -(-NBLK // NW)


def _body(idxT_hbm, ft_hbm, out_hbm, idx_v, acc_v, stg_v, sem):
    wid = lax.axis_index("s") * 2 + lax.axis_index("c")

    def block(k, _):
        blk = k * NW + wid

        @pl.when(blk < NBLK)
        def _():
            base = blk * BB
            pltpu.sync_copy(idxT_hbm.at[blk], idx_v)
            pltpu.async_copy(ft_hbm.at[idx_v.at[0]], acc_v, sem).wait()
            for j in range(1, NUM_FIELDS):
                pltpu.async_copy(ft_hbm.at[idx_v.at[j]], stg_v, sem).wait()

                def row(r, _):
                    for c in range(HIDDEN // 16):
                        sl = pl.ds(c * 16, 16)
                        acc_v[r, sl] = acc_v[r, sl] + stg_v[r, sl]
                    return 0

                lax.fori_loop(0, BB, row, 0)
            pltpu.sync_copy(acc_v, out_hbm.at[pl.ds(base, BB)])

        return 0

    lax.fori_loop(0, BLK_PER_W, block, 0)


@jax.jit
def kernel(x, tables):
    ft = tables.reshape(NUM_FIELDS * VOCAB, HIDDEN)
    offs = (jnp.arange(NUM_FIELDS, dtype=jnp.int32) * VOCAB)[:, None]
    idxT = x.astype(jnp.int32).T + offs
    idx3 = idxT.reshape(NUM_FIELDS, NBLK, BB).transpose(1, 0, 2)

    mesh = plsc.VectorSubcoreMesh(core_axis_name="c", subcore_axis_name="s")
    run = pl.kernel(
        _body,
        out_type=jax.ShapeDtypeStruct((N, HIDDEN), jnp.float32),
        mesh=mesh,
        scratch_types=[
            pltpu.VMEM((NUM_FIELDS, BB), jnp.int32),
            pltpu.VMEM((BB, HIDDEN), jnp.float32),
            pltpu.VMEM((BB, HIDDEN), jnp.float32),
            pltpu.SemaphoreType.DMA,
        ],
    )
    return run(idx3, ft)


# speedup vs baseline: 4.3251x; 1.7357x over previous
import jax
import jax.numpy as jnp
from jax import lax
from jax.experimental import pallas as pl
from jax.experimental.pallas import tpu as pltpu
from jax.experimental.pallas import tpu_sc as plsc

N = 100000
NUM_FIELDS = 9
VOCAB = 100
HIDDEN = 128
NCH = HIDDEN // 16

NW = 32
BB = 80
NBLK = N // BB
BLK_PER_W = -(-NBLK // NW)


def _body(idx_hbm, ft_hbm, out_hbm,
          idx0, idx1, acc0, acc1, s0, s1,
          semA0, semA1, semB0, semB1, semC0, semC1, semI0, semI1):
    wid = lax.axis_index("s") * 2 + lax.axis_index("c")
    idx = (idx0, idx1)
    acc = (acc0, acc1)
    stg = (s0, s1)
    semA = (semA0, semA1)
    semB = (semB0, semB1)
    semC = (semC0, semC1)
    semI = (semI0, semI1)

    def wait_rows(dst, sem):
        pltpu.make_async_copy(ft_hbm.at[idx0.at[0]], dst, sem).wait()

    def add_field(acc_ref, stg_ref):
        @plsc.parallel_loop(0, BB, unroll=2)
        def _(r):
            for c in range(NCH):
                sl = pl.ds(c * 16, 16)
                plsc.addupdate(acc_ref.at[r, sl], stg_ref[r, sl])

    def block(k, p):
        blk = k * NW + wid

        @pl.when(blk < NBLK)
        def _():
            nxt = blk + NW
            has_next = nxt < NBLK
            q = 1 - p

            @pl.when(has_next)
            def _():
                pltpu.async_copy(idx_hbm.at[nxt], idx[q], semI[q])

            wait_rows(acc[p], semA[p])
            for j in range(1, NUM_FIELDS - 1):
                pltpu.async_copy(
                    ft_hbm.at[idx[p].at[j + 1]], stg[(j + 1) % 2],
                    semB[(j + 1) % 2])
                wait_rows(stg[j % 2], semB[j % 2])
                add_field(acc[p], stg[j % 2])

            @pl.when(has_next)
            def _():
                pltpu.make_async_copy(idx_hbm.at[0], idx[q], semI[q]).wait()

                @pl.when(k >= 1)
                def _():
                    wait_rows(acc[q], semC[q])

                pltpu.async_copy(ft_hbm.at[idx[q].at[0]], acc[q], semA[q])
                pltpu.async_copy(ft_hbm.at[idx[q].at[1]], stg[1], semB[1])

            wait_rows(stg[0], semB[0])
            add_field(acc[p], stg[0])
            pltpu.async_copy(acc[p], out_hbm.at[pl.ds(blk * BB, BB)], semC[p])

    pltpu.sync_copy(idx_hbm.at[wid], idx0)
    pltpu.async_copy(ft_hbm.at[idx0.at[0]], acc0, semA0)
    pltpu.async_copy(ft_hbm.at[idx0.at[1]], s1, semB1)

    def pair(kk, _):
        block(2 * kk, 0)
        block(2 * kk + 1, 1)
        return 0

    lax.fori_loop(0, BLK_PER_W // 2, pair, 0)

    wait_rows(acc0, semC0)
    wait_rows(acc1, semC1)


@jax.jit
def kernel(x, tables):
    ft = tables.reshape(NUM_FIELDS * VOCAB, HIDDEN)
    offs = (jnp.arange(NUM_FIELDS, dtype=jnp.int32) * VOCAB)[:, None]
    idxT = x.astype(jnp.int32).T + offs
    idx3 = idxT.reshape(NUM_FIELDS, NBLK, BB).transpose(1, 0, 2)

    mesh = plsc.VectorSubcoreMesh(core_axis_name="c", subcore_axis_name="s")
    run = pl.kernel(
        _body,
        out_type=jax.ShapeDtypeStruct((N, HIDDEN), jnp.float32),
        mesh=mesh,
        scratch_types=[
            pltpu.VMEM((NUM_FIELDS, BB), jnp.int32),
            pltpu.VMEM((NUM_FIELDS, BB), jnp.int32),
            pltpu.VMEM((BB, HIDDEN), jnp.float32),
            pltpu.VMEM((BB, HIDDEN), jnp.float32),
            pltpu.VMEM((BB, HIDDEN), jnp.float32),
            pltpu.VMEM((BB, HIDDEN), jnp.float32),
            pltpu.SemaphoreType.DMA,
            pltpu.SemaphoreType.DMA,
            pltpu.SemaphoreType.DMA,
            pltpu.SemaphoreType.DMA,
            pltpu.SemaphoreType.DMA,
            pltpu.SemaphoreType.DMA,
            pltpu.SemaphoreType.DMA,
            pltpu.SemaphoreType.DMA,
        ],
    )
    return run(idx3, ft)
